# baseline (speedup 1.0000x reference)
import jax
import jax.numpy as jnp
from jax import lax
from jax.experimental import pallas as pl
from jax.experimental.pallas import tpu as pltpu
from jax.experimental.pallas import tpu_sc as plsc

_N = 10000
_E = 320000
_EMB = 128
_OUT_C = 32
_PAD = 144
_NC = 2
_NS = 16
_CHUNK = 64
_NCHUNK = _E // _CHUNK
_NQUAD = _NCHUNK // 4
_RPT = _N // _NS
_BN = 1000


def _prep_body(x_ref, llw_ref, llb_ref, lrw_ref, lrb_ref, alw_ref, alb_ref,
               arw_ref, arb_ref, aq_ref, act_ref, l_ref, rpad_ref, s_ref):
    xb = x_ref[...]
    lb = jnp.dot(xb, llw_ref[...].T, preferred_element_type=jnp.float32) + llb_ref[...][None, :]
    rb = jnp.dot(xb, lrw_ref[...].T, preferred_element_type=jnp.float32) + lrb_ref[...][None, :]
    l_ref[...] = lb
    nrow = xb.shape[0]
    sls, srs = [], []
    for m in range(2):
        a_m = act_ref[0, m]
        al = jnp.dot(lb, alw_ref[m].T, preferred_element_type=jnp.float32) + alb_ref[m][None, :]
        ar = jnp.dot(rb, arw_ref[m].T, preferred_element_type=jnp.float32) + arb_ref[m][None, :]
        q = aq_ref[m, 0]
        sls.append(jnp.sum(jnp.tanh(al) * q[:_OUT_C][None, :], axis=1) * a_m)
        srs.append(jnp.sum(jnp.tanh(ar) * q[_OUT_C:][None, :], axis=1) * a_m)
    rpad_ref[...] = jnp.concatenate(
        [rb, jnp.ones((nrow, 1), jnp.float32),
         srs[0][:, None], srs[1][:, None],
         jnp.zeros((nrow, _PAD - _EMB - 3), jnp.float32)], axis=1)
    s_ref[...] = jnp.stack(sls, axis=1)


def _sc_compute(msgb, slvb, exv, col_l, col_r, lane):
    for g in range(_CHUNK // 16):
        rows = g * 16 + lane
        vl = plsc.load_gather(slvb, [rows, col_l])
        vr = plsc.load_gather(msgb, [rows, col_r])
        exv[pl.ds(g * 16, 16)] = jnp.exp(vl + vr)

    def scale(g, inner):
        e0 = g * 16
        exw = exv[pl.ds(e0, 16)]
        for i in range(16):
            w = exw[i]
            for j in range(_PAD // 16):
                msgb[e0 + i, pl.ds(j * 16, 16)] = (
                    msgb[e0 + i, pl.ds(j * 16, 16)] * w)
        return inner
    lax.fori_loop(0, _CHUNK // 16, scale, 0)


def _sc_body(edges_ref, s_ref, rpad_ref, out_ref,
             slv0, slv1, slv2, slv3, msg0, msg1, msg2, msg3,
             sidxa, didxa, sidxb, didxb, exv, acc,
             sem_g0, sem_g1, sem_g2, sem_g3,
             sem_h0, sem_h1, sem_h2, sem_h3,
             sem_s0, sem_s1, sem_s2, sem_s3):
    c = lax.axis_index("c")
    sid = lax.axis_index("s")
    zero16 = jnp.zeros((16,), jnp.float32)

    def zero_msg(e, carry):
        for j in range(_PAD // 16):
            msg0[e, pl.ds(j * 16, 16)] = zero16
        return carry
    lax.fori_loop(0, _CHUNK, zero_msg, 0)

    r0 = sid * _RPT
    for k in range(_RPT // _CHUNK):
        pltpu.sync_copy(msg0, acc.at[pl.ds(r0 + k * _CHUNK, _CHUNK)])
    _rem = _RPT % _CHUNK
    pltpu.sync_copy(msg0.at[pl.ds(0, _rem)],
                    acc.at[pl.ds(r0 + _RPT - _rem, _rem)])
    plsc.subcore_barrier()

    col_l = jnp.full((16,), c, jnp.int32)
    col_r = jnp.full((16,), _EMB + 1 + c, jnp.int32)
    lane = lax.iota(jnp.int32, 16)
    n_quads = (_NQUAD - sid + _NS - 1) // _NS
    bufs = ((msg0, slv0, sidxa, didxa, 0, sem_g0, sem_h0, sem_s0),
            (msg1, slv1, sidxa, didxa, 1, sem_g1, sem_h1, sem_s1),
            (msg2, slv2, sidxb, didxb, 0, sem_g2, sem_h2, sem_s2),
            (msg3, slv3, sidxb, didxb, 1, sem_g3, sem_h3, sem_s3))

    def drain(k):
        msgb, _, sb, _, h, _, _, sem_s = bufs[k]
        pltpu.make_async_copy(msgb, acc.at[sb.at[0, h]], sem_s).wait()

    def issue_half(j, sidx, didx, bk0):
        pltpu.sync_copy(edges_ref.at[pl.ds(2 * c, 1), pl.ds(j, 2)], sidx)
        pltpu.sync_copy(edges_ref.at[pl.ds(2 * c + 1, 1), pl.ds(j, 2)], didx)
        descs = []
        for k in (bk0, bk0 + 1):
            msgb, slvb, sb, db, h, sem_g, sem_h, _ = bufs[k]
            descs.append(pltpu.async_copy(rpad_ref.at[db.at[0, h]], msgb, sem_g))
            descs.append(pltpu.async_copy(s_ref.at[sb.at[0, h]], slvb, sem_h))
        return descs

    def quad(i, carry):
        q = sid + i * _NS
        j0 = 4 * q

        @pl.when(i > 0)
        def _():
            drain(0)
            drain(1)
        ga = issue_half(j0, sidxa, didxa, 0)

        @pl.when(i > 0)
        def _():
            drain(2)
            drain(3)
        gb = issue_half(j0 + 2, sidxb, didxb, 2)

        for k, descs in ((0, ga[0:2]), (1, ga[2:4]), (2, gb[0:2]), (3, gb[2:4])):
            msgb, slvb, sb, _, h, _, _, sem_s = bufs[k]
            descs[0].wait()
            descs[1].wait()
            _sc_compute(msgb, slvb, exv, col_l, col_r, lane)
            pltpu.async_copy(msgb, acc.at[sb.at[0, h]], sem_s, add=True)
        return carry
    lax.fori_loop(0, n_quads, quad, 0)
    for k in range(4):
        drain(k)

    plsc.subcore_barrier()
    obase = c * _N + r0
    for k in range(_RPT // _CHUNK):
        pltpu.sync_copy(acc.at[pl.ds(r0 + k * _CHUNK, _CHUNK)],
                        out_ref.at[pl.ds(obase + k * _CHUNK, _CHUNK)])
    pltpu.sync_copy(acc.at[pl.ds(r0 + _RPT - _rem, _rem)],
                    out_ref.at[pl.ds(obase + _RPT - _rem, _rem)])


def _combine_body(acc_ref, l_ref, w_ref, o_ref):
    a = acc_ref[...]
    lb = l_ref[...]
    w = w_ref[...][0]
    rel0 = a[0, :, :_EMB] / (a[0, :, _EMB:_EMB + 1] + 1e-16)
    rel1 = a[1, :, :_EMB] / (a[1, :, _EMB:_EMB + 1] + 1e-16)
    t0 = jnp.sum(rel0 * w[None, :], axis=1)
    t1 = jnp.sum(rel1 * w[None, :], axis=1)
    t2 = jnp.sum(lb * w[None, :], axis=1)
    mx = jnp.maximum(jnp.maximum(t0, t1), t2)
    e0 = jnp.exp(t0 - mx)
    e1 = jnp.exp(t1 - mx)
    e2 = jnp.exp(t2 - mx)
    den = e0 + e1 + e2
    o = (e0[:, None] * rel0 + e1[:, None] * rel1 + e2[:, None] * lb) / den[:, None]
    o_ref[...] = jnp.maximum(o, 0.0)


def _full_spec(shape):
    n = len(shape)
    return pl.BlockSpec(shape, lambda i, _n=n: (0,) * _n)


def _run_prep(x_A, lin_l_W, lin_l_b, lin_r_W, lin_r_b, attn_l_W, attn_l_b,
              attn_r_W, attn_r_b, attn_q_W, alpha_act, interpret=False):
    return pl.pallas_call(
        _prep_body,
        grid=(_N // _BN,),
        in_specs=[
            pl.BlockSpec((_BN, _EMB), lambda i: (i, 0)),
            _full_spec((_EMB, _EMB)),
            _full_spec((_EMB,)),
            _full_spec((_EMB, _EMB)),
            _full_spec((_EMB,)),
            _full_spec((2, _OUT_C, _EMB)),
            _full_spec((2, _OUT_C)),
            _full_spec((2, _OUT_C, _EMB)),
            _full_spec((2, _OUT_C)),
            _full_spec((2, 1, 2 * _OUT_C)),
            pl.BlockSpec(memory_space=pltpu.SMEM),
        ],
        out_specs=[
            pl.BlockSpec((_BN, _EMB), lambda i: (i, 0)),
            pl.BlockSpec((_BN, _PAD), lambda i: (i, 0)),
            pl.BlockSpec((_BN, 2), lambda i: (i, 0)),
        ],
        out_shape=[
            jax.ShapeDtypeStruct((_N, _EMB), jnp.float32),
            jax.ShapeDtypeStruct((_N, _PAD), jnp.float32),
            jax.ShapeDtypeStruct((_N, 2), jnp.float32),
        ],
        interpret=interpret,
    )(x_A, lin_l_W, lin_l_b, lin_r_W, lin_r_b, attn_l_W, attn_l_b,
      attn_r_W, attn_r_b, attn_q_W, alpha_act.reshape(1, 2))


def _run_sc(edges4, s_tab, rpad):
    mesh = plsc.VectorSubcoreMesh(core_axis_name="c", subcore_axis_name="s",
                                  num_cores=_NC, num_subcores=_NS)
    return pl.kernel(
        _sc_body,
        out_type=jax.ShapeDtypeStruct((2 * _N, _PAD), jnp.float32),
        mesh=mesh,
        scratch_types=(
            [pltpu.VMEM((_CHUNK, 2), jnp.float32)] * 4
            + [pltpu.VMEM((_CHUNK, _PAD), jnp.float32)] * 4
            + [pltpu.VMEM((1, 2, _CHUNK), jnp.int32)] * 4
            + [pltpu.VMEM((_CHUNK,), jnp.float32)]
            + [pltpu.VMEM_SHARED((_N, _PAD), jnp.float32)]
            + [pltpu.SemaphoreType.DMA] * 12
        ),
        compiler_params=pltpu.CompilerParams(use_tc_tiling_on_sc=False,
                                             needs_layout_passes=False),
    )(edges4, s_tab, rpad)


def _run_combine(acc3, l, conv_W, interpret=False):
    return pl.pallas_call(
        _combine_body,
        grid=(_N // _BN,),
        in_specs=[
            pl.BlockSpec((2, _BN, _PAD), lambda i: (0, i, 0)),
            pl.BlockSpec((_BN, _EMB), lambda i: (i, 0)),
            _full_spec((1, _EMB)),
        ],
        out_specs=pl.BlockSpec((_BN, _EMB), lambda i: (i, 0)),
        out_shape=jax.ShapeDtypeStruct((_N, _EMB), jnp.float32),
        interpret=interpret,
    )(acc3, l, conv_W)


def kernel(x_A, edge_index_r0, edge_index_r1, global_node_idx_A, lin_l_W,
           lin_l_b, lin_r_W, lin_r_b, attn_l_W, attn_l_b, attn_r_W, attn_r_b,
           attn_q_W, conv_W, conv_b, alpha_act):
    l, rpad, s_tab = _run_prep(x_A, lin_l_W, lin_l_b, lin_r_W, lin_r_b,
                               attn_l_W, attn_l_b, attn_r_W, attn_r_b,
                               attn_q_W, alpha_act)
    edges4 = jnp.concatenate([edge_index_r0, edge_index_r1],
                             axis=0).reshape(4, _NCHUNK, _CHUNK)
    acc = _run_sc(edges4, s_tab, rpad)
    acc3 = acc.reshape(2, _N, _PAD)
    return _run_combine(acc3, l, conv_W)


# speedup vs baseline: 1.1742x; 1.1742x over previous
import jax
import jax.numpy as jnp
from jax import lax
from jax.experimental import pallas as pl
from jax.experimental.pallas import tpu as pltpu
from jax.experimental.pallas import tpu_sc as plsc

_N = 10000
_E = 320000
_EMB = 128
_OUT_C = 32
_PAD = 144
_NC = 2
_NS = 16
_CHUNK = 64
_NCHUNK = _E // _CHUNK
_NPAIR = _NCHUNK // 2
_NQUAD = _NCHUNK // 4
_RPT = _N // _NS
_BN = 1000


def _prep_body(x_ref, llw_ref, llb_ref, lrw_ref, lrb_ref,
               arw_ref, arb_ref, aq_ref, act_ref, l_ref, rp_ref):
    xb = x_ref[...]
    lb = jnp.dot(xb, llw_ref[...].T, preferred_element_type=jnp.float32) + llb_ref[...][None, :]
    rb = jnp.dot(xb, lrw_ref[...].T, preferred_element_type=jnp.float32) + lrb_ref[...][None, :]
    l_ref[...] = lb
    nrow = xb.shape[0]
    zpad = jnp.zeros((nrow, _PAD - _EMB - 1), jnp.float32)
    for m in range(2):
        a_m = act_ref[0, m]
        ar = jnp.dot(rb, arw_ref[m].T, preferred_element_type=jnp.float32) + arb_ref[m][None, :]
        q = aq_ref[m, 0]
        esr = jnp.exp(jnp.sum(jnp.tanh(ar) * q[_OUT_C:][None, :], axis=1) * a_m)
        rp_ref[m] = jnp.concatenate(
            [rb * esr[:, None], esr[:, None], zpad], axis=1)


def _sc_body(edges_ref, rp_ref, out_ref,
             msg0, msg1, msg2, msg3, idxa, idxb, acc,
             sem_g0, sem_g1, sem_g2, sem_g3,
             sem_s0, sem_s1, sem_s2, sem_s3):
    c = lax.axis_index("c")
    sid = lax.axis_index("s")
    zero16 = jnp.zeros((16,), jnp.float32)

    def zero_msg(e, carry):
        for j in range(_PAD // 16):
            msg0[e, pl.ds(j * 16, 16)] = zero16
        return carry
    lax.fori_loop(0, _CHUNK, zero_msg, 0)

    r0 = sid * _RPT
    for k in range(_RPT // _CHUNK):
        pltpu.sync_copy(msg0, acc.at[pl.ds(r0 + k * _CHUNK, _CHUNK)])
    _rem = _RPT % _CHUNK
    pltpu.sync_copy(msg0.at[pl.ds(0, _rem)],
                    acc.at[pl.ds(r0 + _RPT - _rem, _rem)])
    plsc.subcore_barrier()

    n_quads = (_NQUAD - sid + _NS - 1) // _NS
    bufs = ((msg0, idxa, 0, sem_g0, sem_s0),
            (msg1, idxa, 1, sem_g1, sem_s1),
            (msg2, idxb, 0, sem_g2, sem_s2),
            (msg3, idxb, 1, sem_g3, sem_s3))

    def drain(k):
        msgb, idx, h, _, sem_s = bufs[k]
        pltpu.make_async_copy(msgb, acc.at[idx.at[0, 0, h, 0]], sem_s).wait()

    def issue_half(p, idx, bk0):
        pltpu.sync_copy(edges_ref.at[pl.ds(c, 1), pl.ds(p, 1)], idx)
        descs = []
        for k in (bk0, bk0 + 1):
            msgb, _, h, sem_g, _ = bufs[k]
            descs.append(
                pltpu.async_copy(rp_ref.at[idx.at[0, 0, h, 1]], msgb, sem_g))
        return descs

    def quad(i, carry):
        q = sid + i * _NS

        @pl.when(i > 0)
        def _():
            drain(0)
            drain(1)
        ga = issue_half(2 * q, idxa, 0)

        @pl.when(i > 0)
        def _():
            drain(2)
            drain(3)
        gb = issue_half(2 * q + 1, idxb, 2)

        for k, desc in ((0, ga[0]), (1, ga[1]), (2, gb[0]), (3, gb[1])):
            msgb, idx, h, _, sem_s = bufs[k]
            desc.wait()
            pltpu.async_copy(msgb, acc.at[idx.at[0, 0, h, 0]], sem_s, add=True)
        return carry
    lax.fori_loop(0, n_quads, quad, 0)
    for k in range(4):
        drain(k)

    plsc.subcore_barrier()
    obase = c * _N + r0
    for k in range(_RPT // _CHUNK):
        pltpu.sync_copy(acc.at[pl.ds(r0 + k * _CHUNK, _CHUNK)],
                        out_ref.at[pl.ds(obase + k * _CHUNK, _CHUNK)])
    pltpu.sync_copy(acc.at[pl.ds(r0 + _RPT - _rem, _rem)],
                    out_ref.at[pl.ds(obase + _RPT - _rem, _rem)])


def _combine_body(acc_ref, l_ref, w_ref, o_ref):
    a = acc_ref[...]
    lb = l_ref[...]
    w = w_ref[...][0]
    rel0 = a[0, :, :_EMB] / (a[0, :, _EMB:_EMB + 1] + 1e-16)
    rel1 = a[1, :, :_EMB] / (a[1, :, _EMB:_EMB + 1] + 1e-16)
    t0 = jnp.sum(rel0 * w[None, :], axis=1)
    t1 = jnp.sum(rel1 * w[None, :], axis=1)
    t2 = jnp.sum(lb * w[None, :], axis=1)
    mx = jnp.maximum(jnp.maximum(t0, t1), t2)
    e0 = jnp.exp(t0 - mx)
    e1 = jnp.exp(t1 - mx)
    e2 = jnp.exp(t2 - mx)
    den = e0 + e1 + e2
    o = (e0[:, None] * rel0 + e1[:, None] * rel1 + e2[:, None] * lb) / den[:, None]
    o_ref[...] = jnp.maximum(o, 0.0)


def _full_spec(shape):
    n = len(shape)
    return pl.BlockSpec(shape, lambda i, _n=n: (0,) * _n)


def _run_prep(x_A, lin_l_W, lin_l_b, lin_r_W, lin_r_b,
              attn_r_W, attn_r_b, attn_q_W, alpha_act, interpret=False):
    return pl.pallas_call(
        _prep_body,
        grid=(_N // _BN,),
        in_specs=[
            pl.BlockSpec((_BN, _EMB), lambda i: (i, 0)),
            _full_spec((_EMB, _EMB)),
            _full_spec((_EMB,)),
            _full_spec((_EMB, _EMB)),
            _full_spec((_EMB,)),
            _full_spec((2, _OUT_C, _EMB)),
            _full_spec((2, _OUT_C)),
            _full_spec((2, 1, 2 * _OUT_C)),
            pl.BlockSpec(memory_space=pltpu.SMEM),
        ],
        out_specs=[
            pl.BlockSpec((_BN, _EMB), lambda i: (i, 0)),
            pl.BlockSpec((2, _BN, _PAD), lambda i: (0, i, 0)),
        ],
        out_shape=[
            jax.ShapeDtypeStruct((_N, _EMB), jnp.float32),
            jax.ShapeDtypeStruct((2, _N, _PAD), jnp.float32),
        ],
        interpret=interpret,
    )(x_A, lin_l_W, lin_l_b, lin_r_W, lin_r_b,
      attn_r_W, attn_r_b, attn_q_W, alpha_act.reshape(1, 2))


def _run_sc(edges5, rp2):
    mesh = plsc.VectorSubcoreMesh(core_axis_name="c", subcore_axis_name="s",
                                  num_cores=_NC, num_subcores=_NS)
    return pl.kernel(
        _sc_body,
        out_type=jax.ShapeDtypeStruct((2 * _N, _PAD), jnp.float32),
        mesh=mesh,
        scratch_types=(
            [pltpu.VMEM((_CHUNK, _PAD), jnp.float32)] * 4
            + [pltpu.VMEM((1, 1, 2, 2, _CHUNK), jnp.int32)] * 2
            + [pltpu.VMEM_SHARED((_N, _PAD), jnp.float32)]
            + [pltpu.SemaphoreType.DMA] * 8
        ),
        compiler_params=pltpu.CompilerParams(use_tc_tiling_on_sc=False,
                                             needs_layout_passes=False),
    )(edges5, rp2)


def _run_combine(acc3, l, conv_W, interpret=False):
    return pl.pallas_call(
        _combine_body,
        grid=(_N // _BN,),
        in_specs=[
            pl.BlockSpec((2, _BN, _PAD), lambda i: (0, i, 0)),
            pl.BlockSpec((_BN, _EMB), lambda i: (i, 0)),
            _full_spec((1, _EMB)),
        ],
        out_specs=pl.BlockSpec((_BN, _EMB), lambda i: (i, 0)),
        out_shape=jax.ShapeDtypeStruct((_N, _EMB), jnp.float32),
        interpret=interpret,
    )(acc3, l, conv_W)


def _pack_edges(edge_index_r0, edge_index_r1):
    rows = []
    for m, ei in enumerate((edge_index_r0, edge_index_r1)):
        src = ei[0].reshape(_NCHUNK, 1, _CHUNK)
        dst = (ei[1] + m * _N).reshape(_NCHUNK, 1, _CHUNK)
        rows.append(jnp.concatenate([src, dst], axis=1))
    return jnp.stack(rows).reshape(2, _NPAIR, 2, 2, _CHUNK)


def kernel(x_A, edge_index_r0, edge_index_r1, global_node_idx_A, lin_l_W,
           lin_l_b, lin_r_W, lin_r_b, attn_l_W, attn_l_b, attn_r_W, attn_r_b,
           attn_q_W, conv_W, conv_b, alpha_act):
    l, rp2 = _run_prep(x_A, lin_l_W, lin_l_b, lin_r_W, lin_r_b,
                       attn_r_W, attn_r_b, attn_q_W, alpha_act)
    edges5 = _pack_edges(edge_index_r0, edge_index_r1)
    acc = _run_sc(edges5, rp2.reshape(2 * _N, _PAD))
    acc3 = acc.reshape(2, _N, _PAD)
    return _run_combine(acc3, l, conv_W)
